# baseline (speedup 1.0000x reference)
import functools

import jax
import jax.numpy as jnp
from jax import lax
from jax.experimental import pallas as pl
from jax.experimental.pallas import tpu as pltpu
from jax.experimental.pallas import tpu_sc as plsc

NC = 2
NS = 16
LANE = 16


def _signal_tc(y, Wt, b):
    B, S = y.shape
    D = Wt.shape[1]
    blk = 2048

    def body(y_ref, wt_ref, b_ref, o_ref):
        o_ref[...] = (
            lax.dot_general(
                y_ref[...], wt_ref[...],
                (((1,), (0,)), ((), ())),
                preferred_element_type=jnp.float32,
            )
            + b_ref[...]
        )

    return pl.pallas_call(
        body,
        grid=(B // blk,),
        in_specs=[
            pl.BlockSpec((blk, S), lambda i: (i, 0)),
            pl.BlockSpec((S, D), lambda i: (0, 0)),
            pl.BlockSpec((1, D), lambda i: (0, 0)),
        ],
        out_specs=pl.BlockSpec((blk, D), lambda i: (i, 0)),
        out_shape=jax.ShapeDtypeStruct((B, D), jnp.float32),
    )(y, Wt, b.reshape(1, D))


def _make_sc_kernel(B, L, D, bpw, idx_chunk, nbuf):
    n_chunks = L // idx_chunk
    mesh = plsc.VectorSubcoreMesh(core_axis_name="c", subcore_axis_name="s")

    @functools.partial(
        pl.kernel,
        mesh=mesh,
        out_type=jax.ShapeDtypeStruct((B, L, D), jnp.float32),
        scratch_types=[
            pltpu.VMEM((nbuf, n_chunks, idx_chunk), jnp.int32),
            pltpu.VMEM((nbuf, L, D), jnp.float32),
            pltpu.VMEM((bpw, D), jnp.float32),
            [pltpu.SemaphoreType.DMA] * nbuf,
            [pltpu.SemaphoreType.DMA] * nbuf,
        ],
        compiler_params=pltpu.CompilerParams(use_tc_tiling_on_sc=False),
    )
    def sc_k(x_hbm, sig_hbm, emb_hbm, out_hbm, idx_v, rows_v, sig_v,
             gsems, wsems):
        wid = lax.axis_index("s") * NC + lax.axis_index("c")
        batch0 = wid * bpw
        pltpu.sync_copy(sig_hbm.at[pl.ds(batch0, bpw)], sig_v)

        def gather_start(i, b):
            batch = batch0 + i
            pltpu.sync_copy(
                x_hbm.at[pl.ds(batch * n_chunks, n_chunks)], idx_v.at[b])
            for c in range(n_chunks):
                pltpu.async_copy(
                    emb_hbm.at[idx_v.at[b].at[c]],
                    rows_v.at[b].at[pl.ds(c * idx_chunk, idx_chunk)],
                    gsems[b],
                )

        def gather_wait(b):
            for c in range(n_chunks):
                pltpu.make_async_copy(
                    emb_hbm.at[idx_v.at[b].at[c]],
                    rows_v.at[b].at[pl.ds(c * idx_chunk, idx_chunk)],
                    gsems[b],
                ).wait()

        def write_start(i, b):
            batch = batch0 + i
            pltpu.async_copy(rows_v.at[b], out_hbm.at[batch], wsems[b])

        def write_wait(b):
            pltpu.make_async_copy(
                rows_v.at[b], out_hbm.at[batch0], wsems[b]).wait()

        gather_start(0, 0)

        def body(h, carry):
            for b in range(nbuf):
                i = h * nbuf + b
                j = i + 1
                nb = (b + 1) % nbuf

                @pl.when(j < bpw)
                def _():
                    @pl.when(j >= nbuf)
                    def _():
                        write_wait(nb)

                    gather_start(j, nb)

                gather_wait(b)
                sig = [sig_v[i, pl.ds(k * LANE, LANE)]
                       for k in range(D // LANE)]

                def row_body(r, c2):
                    for k in range(D // LANE):
                        rows_v[b, r, pl.ds(k * LANE, LANE)] += sig[k]
                    return c2

                lax.fori_loop(0, L, row_body, 0, unroll=4)
                write_start(i, b)
            return carry

        lax.fori_loop(0, bpw // nbuf, body, 0)
        for b in range(nbuf):
            write_wait(b)

    return sc_k


def kernel(x, y, embedding, W, b):
    B, L = x.shape
    V, D = embedding.shape
    NW = NC * NS
    bpw = B // NW
    idx_chunk = 100
    x2d = x.reshape(B * L // idx_chunk, idx_chunk).astype(jnp.int32)
    Wt = W.T.astype(jnp.float32)
    sig = _signal_tc(y.astype(jnp.float32), Wt, b.astype(jnp.float32))
    sc_k = _make_sc_kernel(B, L, D, bpw, idx_chunk, nbuf=4)
    return sc_k(x2d, sig, embedding)


# speedup vs baseline: 1.6192x; 1.6192x over previous
import functools

import jax
import jax.numpy as jnp
from jax import lax
from jax.experimental import pallas as pl
from jax.experimental.pallas import tpu as pltpu
from jax.experimental.pallas import tpu_sc as plsc

NC = 2
NS = 16
LANE = 16


def _signal_tc(y, Wt, b):
    B, S = y.shape
    D = Wt.shape[1]
    blk = 2048

    def body(y_ref, wt_ref, b_ref, o_ref):
        o_ref[...] = (
            lax.dot_general(
                y_ref[...], wt_ref[...],
                (((1,), (0,)), ((), ())),
                preferred_element_type=jnp.float32,
            )
            + b_ref[...]
        )

    return pl.pallas_call(
        body,
        grid=(B // blk,),
        in_specs=[
            pl.BlockSpec((blk, S), lambda i: (i, 0)),
            pl.BlockSpec((S, D), lambda i: (0, 0)),
            pl.BlockSpec((1, D), lambda i: (0, 0)),
        ],
        out_specs=pl.BlockSpec((blk, D), lambda i: (i, 0)),
        out_shape=jax.ShapeDtypeStruct((B, D), jnp.float32),
    )(y, Wt, b.reshape(1, D))


def _make_sc_kernel(B, L, D, bpw, idx_chunk, nbuf):
    n_chunks = L // idx_chunk
    mesh = plsc.VectorSubcoreMesh(core_axis_name="c", subcore_axis_name="s")

    @functools.partial(
        pl.kernel,
        mesh=mesh,
        out_type=jax.ShapeDtypeStruct((B, L, 2 * D), jnp.float32),
        scratch_types=[
            pltpu.VMEM((nbuf, n_chunks, idx_chunk), jnp.int32),
            pltpu.VMEM((nbuf, L, D), jnp.float32),
            pltpu.VMEM((bpw, D), jnp.float32),
            [pltpu.SemaphoreType.DMA] * nbuf,
            [pltpu.SemaphoreType.DMA] * nbuf,
        ],
        compiler_params=pltpu.CompilerParams(use_tc_tiling_on_sc=False),
    )
    def sc_k(x_hbm, sig_hbm, emb_hbm, out_hbm, idx_v, rows_v, sig_v,
             gsems, wsems):
        wid = lax.axis_index("s") * NC + lax.axis_index("c")
        batch0 = wid * bpw
        pltpu.sync_copy(sig_hbm.at[pl.ds(batch0, bpw)], sig_v)

        def gather_start(i, b):
            batch = batch0 + i
            pltpu.sync_copy(
                x_hbm.at[pl.ds(batch * n_chunks, n_chunks)], idx_v.at[b])
            for c in range(n_chunks):
                pltpu.async_copy(
                    emb_hbm.at[idx_v.at[b].at[c]],
                    rows_v.at[b].at[pl.ds(c * idx_chunk, idx_chunk)],
                    gsems[b],
                )

        def gather_wait(b):
            for c in range(n_chunks):
                pltpu.make_async_copy(
                    emb_hbm.at[idx_v.at[b].at[c]],
                    rows_v.at[b].at[pl.ds(c * idx_chunk, idx_chunk)],
                    gsems[b],
                ).wait()

        def write_start(i, b):
            batch = batch0 + i
            pltpu.async_copy(
                rows_v.at[b], out_hbm.at[batch, :, pl.ds(0, D)], wsems[b])

        def write_wait(b):
            pltpu.make_async_copy(
                rows_v.at[b], out_hbm.at[batch0, :, pl.ds(0, D)],
                wsems[b]).wait()

        gather_start(0, 0)

        def body(h, carry):
            for b in range(nbuf):
                i = h * nbuf + b
                j = i + 1
                nb = (b + 1) % nbuf

                @pl.when(j < bpw)
                def _():
                    @pl.when(j >= nbuf)
                    def _():
                        write_wait(nb)

                    gather_start(j, nb)

                gather_wait(b)
                sig = [sig_v[i, pl.ds(k * LANE, LANE)]
                       for k in range(D // LANE)]

                def row_body(r, c2):
                    for k in range(D // LANE):
                        rows_v[b, r, pl.ds(k * LANE, LANE)] += sig[k]
                    return c2

                lax.fori_loop(0, L, row_body, 0, unroll=4)
                write_start(i, b)
            return carry

        lax.fori_loop(0, bpw // nbuf, body, 0)
        for b in range(nbuf):
            write_wait(b)

    return sc_k


def kernel(x, y, embedding, W, b):
    B, L = x.shape
    V, D = embedding.shape
    NW = NC * NS
    bpw = B // NW
    idx_chunk = 100
    x2d = x.reshape(B * L // idx_chunk, idx_chunk).astype(jnp.int32)
    Wt = W.T.astype(jnp.float32)
    sig = _signal_tc(y.astype(jnp.float32), Wt, b.astype(jnp.float32))
    sc_k = _make_sc_kernel(B, L, D, bpw, idx_chunk, nbuf=4)
    return sc_k(x2d, sig, embedding)[:, :, :D]


# speedup vs baseline: 1.6679x; 1.0300x over previous
import functools

import jax
import jax.numpy as jnp
from jax import lax
from jax.experimental import pallas as pl
from jax.experimental.pallas import tpu as pltpu
from jax.experimental.pallas import tpu_sc as plsc

NC = 2
NS = 16
LANE = 16


def _signal_tc(y, Wt, b):
    B, S = y.shape
    D = Wt.shape[1]
    blk = 2048

    def body(y_ref, wt_ref, b_ref, o_ref):
        o_ref[...] = (
            lax.dot_general(
                y_ref[...], wt_ref[...],
                (((1,), (0,)), ((), ())),
                preferred_element_type=jnp.float32,
            )
            + b_ref[...]
        )

    return pl.pallas_call(
        body,
        grid=(B // blk,),
        in_specs=[
            pl.BlockSpec((blk, S), lambda i: (i, 0)),
            pl.BlockSpec((S, D), lambda i: (0, 0)),
            pl.BlockSpec((1, D), lambda i: (0, 0)),
        ],
        out_specs=pl.BlockSpec((blk, D), lambda i: (i, 0)),
        out_shape=jax.ShapeDtypeStruct((B, D), jnp.float32),
    )(y, Wt, b.reshape(1, D))


def _make_sc_kernel(B, L, D, bpw, idx_chunk, nbuf):
    n_chunks = L // idx_chunk
    mesh = plsc.VectorSubcoreMesh(core_axis_name="c", subcore_axis_name="s")

    @functools.partial(
        pl.kernel,
        mesh=mesh,
        out_type=jax.ShapeDtypeStruct((B, L, 2 * D), jnp.float32),
        scratch_types=[
            pltpu.VMEM((nbuf, n_chunks, idx_chunk), jnp.int32),
            pltpu.VMEM((nbuf, L, D), jnp.float32),
            pltpu.VMEM((bpw, D), jnp.float32),
            [pltpu.SemaphoreType.DMA] * nbuf,
            [pltpu.SemaphoreType.DMA] * nbuf,
        ],
        compiler_params=pltpu.CompilerParams(use_tc_tiling_on_sc=False),
    )
    def sc_k(x_hbm, sig_hbm, emb_hbm, out_hbm, idx_v, rows_v, sig_v,
             gsems, wsems):
        wid = lax.axis_index("s") * NC + lax.axis_index("c")
        batch0 = wid * bpw
        pltpu.sync_copy(sig_hbm.at[pl.ds(batch0, bpw)], sig_v)

        def gather_start(i, b):
            batch = batch0 + i
            pltpu.sync_copy(
                x_hbm.at[pl.ds(batch * n_chunks, n_chunks)], idx_v.at[b])
            for c in range(n_chunks):
                pltpu.async_copy(
                    emb_hbm.at[idx_v.at[b].at[c]],
                    rows_v.at[b].at[pl.ds(c * idx_chunk, idx_chunk)],
                    gsems[b],
                )

        def gather_wait(b):
            for c in range(n_chunks):
                pltpu.make_async_copy(
                    emb_hbm.at[idx_v.at[b].at[c]],
                    rows_v.at[b].at[pl.ds(c * idx_chunk, idx_chunk)],
                    gsems[b],
                ).wait()

        def write_start(i, b):
            batch = batch0 + i
            pltpu.async_copy(
                rows_v.at[b], out_hbm.at[batch, :, pl.ds(0, D)], wsems[b])

        def write_wait(b):
            pltpu.make_async_copy(
                rows_v.at[b], out_hbm.at[batch0, :, pl.ds(0, D)],
                wsems[b]).wait()

        gather_start(0, 0)

        def body(h, carry):
            for b in range(nbuf):
                i = h * nbuf + b
                j = i + 1
                nb = (b + 1) % nbuf

                @pl.when(j < bpw)
                def _():
                    @pl.when(j >= nbuf)
                    def _():
                        write_wait(nb)

                    gather_start(j, nb)

                gather_wait(b)
                sig = [sig_v[i, pl.ds(k * LANE, LANE)]
                       for k in range(D // LANE)]

                def row_body(r, c2):
                    for k in range(D // LANE):
                        rows_v[b, r, pl.ds(k * LANE, LANE)] += sig[k]
                    return c2

                lax.fori_loop(0, L, row_body, 0, unroll=4)
                write_start(i, b)
            return carry

        lax.fori_loop(0, bpw // nbuf, body, 0)
        for b in range(nbuf):
            write_wait(b)

    return sc_k


def kernel(x, y, embedding, W, b):
    B, L = x.shape
    V, D = embedding.shape
    NW = NC * NS
    bpw = B // NW
    idx_chunk = 100
    x2d = (x.reshape(B * L // idx_chunk, idx_chunk).astype(jnp.int32) * 2)
    emb2 = jnp.pad(embedding, ((0, 0), (0, D))).reshape(2 * V, D)
    Wt = W.T.astype(jnp.float32)
    sig = _signal_tc(y.astype(jnp.float32), Wt, b.astype(jnp.float32))
    sc_k = _make_sc_kernel(B, L, D, bpw, idx_chunk, nbuf=4)
    return sc_k(x2d, sig, emb2)[:, :, :D]
